# baseline (speedup 1.0000x reference)
import functools

import jax
import jax.numpy as jnp
from jax import lax
from jax.experimental import pallas as pl
from jax.experimental.pallas import tpu as pltpu
from jax.experimental.pallas import tpu_sc as plsc

N = 10000
G = 16
D = 128
EPS = 1e-5

NC = 2
NS = 16
NW = NC * NS
C = 128

ROWS = 10240
RPT = ROWS // NS

_mesh = plsc.VectorSubcoreMesh(core_axis_name="c", subcore_axis_name="s")


def _deg_body(edges, zeros1d, degs, hist, sh, acc, tmp, idx):
    nch2 = edges.shape[2]
    cid = lax.axis_index("c")
    sid = lax.axis_index("s")
    r0 = sid * RPT
    pltpu.sync_copy(zeros1d, hist)
    pltpu.sync_copy(edges.at[cid, sid], idx)
    ones = jnp.ones((16,), jnp.float32)

    def body(j, carry):
        for i in range(C // 16):
            plsc.addupdate_scatter(hist, [idx[j, pl.ds(i * 16, 16)]], ones)
        return carry

    lax.fori_loop(0, nch2, body, 0)

    pltpu.sync_copy(hist, sh.at[sid])
    plsc.subcore_barrier()

    pltpu.sync_copy(sh.at[0, pl.ds(r0, RPT)], acc)

    def red(k, carry):
        pltpu.sync_copy(sh.at[k, pl.ds(r0, RPT)], tmp)
        for i in range(RPT // 16):
            s = pl.ds(i * 16, 16)
            acc[s] = acc[s] + tmp[s]
        return carry

    lax.fori_loop(1, NS, red, 0)
    pltpu.sync_copy(acc, degs.at[cid, 0, pl.ds(r0, RPT)])


def _sc_degrees(edges):
    zeros1d = jnp.zeros((ROWS,), jnp.float32)
    out_type = jax.ShapeDtypeStruct((NC, 1, ROWS), jnp.float32)
    scratch = [
        pltpu.VMEM((ROWS,), jnp.float32),
        pltpu.VMEM_SHARED((NS, ROWS), jnp.float32),
        pltpu.VMEM((RPT,), jnp.float32),
        pltpu.VMEM((RPT,), jnp.float32),
        pltpu.VMEM((edges.shape[2], C), jnp.int32),
    ]
    fn = pl.kernel(_deg_body, out_type=out_type, mesh=_mesh,
                   scratch_types=scratch,
                   compiler_params=pltpu.CompilerParams(
                       needs_layout_passes=False))
    return fn(edges, zeros1d)


NBUF = 2


def _agg_body(xs, srcg, dstp, zrows, out, agg_sh, idx_s, idx_d,
              rows, sis, sid_, sg, ss):
    nch = srcg.shape[2]
    cid = lax.axis_index("c")
    sid = lax.axis_index("s")
    r0 = sid * RPT
    pltpu.sync_copy(zrows, agg_sh.at[pl.ds(r0, RPT)])
    plsc.subcore_barrier()

    def body(j, carry):
        pltpu.sync_copy(srcg.at[cid, sid, j], idx_s[0])
        pltpu.sync_copy(dstp.at[cid, sid, j], idx_d[0])
        pltpu.async_copy(xs.at[idx_s[0]], rows[0], sg[0]).wait()
        pltpu.sync_copy(rows[0], agg_sh.at[idx_d[0]], add=True)
        return carry

    lax.fori_loop(0, nch, body, 0)

    plsc.subcore_barrier()
    pltpu.sync_copy(agg_sh.at[pl.ds(r0, RPT)], out.at[cid, pl.ds(r0, RPT)])


def _sc_aggregate(xs, srcg, dstp):
    zrows = jnp.zeros((RPT, D), jnp.float32)
    out_type = jax.ShapeDtypeStruct((NC, ROWS, D), jnp.float32)
    scratch = [
        pltpu.VMEM_SHARED((ROWS, D), jnp.float32),
        [pltpu.VMEM((C,), jnp.int32)] * 2,
        [pltpu.VMEM((C,), jnp.int32)] * 2,
        [pltpu.VMEM((C, D), jnp.float32)] * 2,
        [pltpu.SemaphoreType.DMA] * 2,
        [pltpu.SemaphoreType.DMA] * 2,
        [pltpu.SemaphoreType.DMA] * 2,
        [pltpu.SemaphoreType.DMA] * 2,
    ]
    fn = pl.kernel(_agg_body, out_type=out_type, mesh=_mesh,
                   scratch_types=scratch)
    return fn(xs, srcg, dstp, zrows)


def _norm_col(deg):
    return jnp.where(deg > 0, lax.rsqrt(jnp.maximum(deg, 1.0)), 0.0)


def _tc_a_body(h_ref, w_ref, dego_ref, xs_ref):
    ns = _norm_col(dego_ref[...])
    x = lax.dot_general(h_ref[...], w_ref[...], (((1,), (1,)), ((), ())),
                        preferred_element_type=jnp.float32)
    xs_ref[...] = x * ns


def _tc_a(h, weights, dego):
    return pl.pallas_call(
        _tc_a_body,
        out_shape=jax.ShapeDtypeStruct((N, D), jnp.float32),
    )(h, weights, dego)


def _layer_post(aggp, degi, W, b, g, beta):
    nd = _norm_col(degi)
    agg = (aggp[0, :N, :] + aggp[1, :N, :]) * nd
    t = lax.dot_general(agg, W, (((1,), (0,)), ((), ())),
                        preferred_element_type=jnp.float32) + b
    m = jnp.mean(t, axis=0, keepdims=True)
    v = jnp.mean((t - m) ** 2, axis=0, keepdims=True)
    t = (t - m) * lax.rsqrt(v + EPS) * g + beta
    return jnp.maximum(t, 0.0)


def _tc_b_body(aggp_ref, dego_ref, degi_ref, W_ref, b_ref, g_ref, beta_ref,
               xs_ref):
    r = _layer_post(aggp_ref[...], degi_ref[...], W_ref[...], b_ref[...],
                    g_ref[...], beta_ref[...])
    xs_ref[...] = r * _norm_col(dego_ref[...])


def _tc_b(aggp, dego, degi, W, b, g, beta):
    return pl.pallas_call(
        _tc_b_body,
        out_shape=jax.ShapeDtypeStruct((N, D), jnp.float32),
    )(aggp, dego, degi, W, b, g, beta)


def _tc_c_body(aggp_ref, degi_ref, W_ref, b_ref, g_ref, beta_ref, gid_ref,
               fcW_ref, fcb_ref, out_ref):
    r = _layer_post(aggp_ref[...], degi_ref[...], W_ref[...], b_ref[...],
                    g_ref[...], beta_ref[...])
    gid = gid_ref[...]
    neg = jnp.float32(-jnp.inf)
    rows = [jnp.max(jnp.where(gid == gg, r, neg), axis=0, keepdims=True)
            for gg in range(G)]
    pooled = jnp.concatenate(rows, axis=0)
    out_ref[...] = lax.dot_general(pooled, fcW_ref[...],
                                   (((1,), (1,)), ((), ())),
                                   preferred_element_type=jnp.float32) \
        + fcb_ref[...]


def _tc_c(aggp, degi, W, b, g, beta, gid, fcW, fcb):
    return pl.pallas_call(
        _tc_c_body,
        out_shape=jax.ShapeDtypeStruct((G, D), jnp.float32),
    )(aggp, degi, W, b, g, beta, gid, fcW, fcb)


def kernel(h, edge_index, graph_ids, weights, W0, b0, g0, beta0,
           W1, b1, g1, beta1, fcW, fcb):
    src = edge_index[0]
    dst = edge_index[1]
    E = src.shape[0]
    ept = E // NW
    nch = -(-ept // (C * NBUF)) * NBUF
    pad = nch * C - ept

    srcm = src.reshape(NW, ept)
    dstm = dst.reshape(NW, ept)
    shape4 = (NC, NS, nch, C)
    srcg = jnp.pad(srcm, ((0, 0), (0, pad))).reshape(shape4)
    srcd = jnp.pad(srcm, ((0, 0), (0, pad)),
                   constant_values=N).reshape(shape4)
    dstp = jnp.pad(dstm, ((0, 0), (0, pad)),
                   constant_values=N).reshape(shape4)

    b0r, g0r, beta0r = b0.reshape(1, -1), g0.reshape(1, -1), beta0.reshape(1, -1)
    b1r, g1r, beta1r = b1.reshape(1, -1), g1.reshape(1, -1), beta1.reshape(1, -1)
    fcbr = fcb.reshape(1, -1)
    gid = graph_ids.reshape(N, 1)

    edges_deg = jnp.stack([srcd.reshape(NS, 2 * nch, C),
                           dstp.reshape(NS, 2 * nch, C)])
    degs = _sc_degrees(edges_deg)
    dego = degs[0, 0, :N].reshape(N, 1)
    degi = degs[1, 0, :N].reshape(N, 1)
    xs1 = _tc_a(h, weights, dego)
    aggp1 = _sc_aggregate(xs1, srcg, dstp)
    xs2 = _tc_b(aggp1, dego, degi, W0, b0r, g0r, beta0r)
    aggp2 = _sc_aggregate(xs2, srcg, dstp)
    out = _tc_c(aggp2, degi, W1, b1r, g1r, beta1r, gid, fcW, fcbr)
    return out


# speedup vs baseline: 1.2588x; 1.2588x over previous
import functools

import jax
import jax.numpy as jnp
from jax import lax
from jax.experimental import pallas as pl
from jax.experimental.pallas import tpu as pltpu
from jax.experimental.pallas import tpu_sc as plsc

N = 10000
G = 16
D = 128
EPS = 1e-5

NC = 2
NS = 16
NW = NC * NS
C = 128

ROWS = 10240
RPT = ROWS // NS

_mesh = plsc.VectorSubcoreMesh(core_axis_name="c", subcore_axis_name="s")


def _deg_body(edges, zeros1d, degs, hist, sh, acc, tmp, idx):
    nch2 = edges.shape[2]
    cid = lax.axis_index("c")
    sid = lax.axis_index("s")
    r0 = sid * RPT
    pltpu.sync_copy(zeros1d, hist)
    ones = jnp.ones((16,), jnp.float32)

    def body(j, carry):
        pltpu.sync_copy(edges.at[cid, sid, j], idx)
        for i in range(C // 16):
            plsc.addupdate_scatter(hist, [idx[pl.ds(i * 16, 16)]], ones)
        return carry

    lax.fori_loop(0, nch2, body, 0)

    pltpu.sync_copy(hist, sh.at[sid])
    plsc.subcore_barrier()

    pltpu.sync_copy(sh.at[0, pl.ds(r0, RPT)], acc)

    def red(k, carry):
        pltpu.sync_copy(sh.at[k, pl.ds(r0, RPT)], tmp)
        for i in range(RPT // 16):
            s = pl.ds(i * 16, 16)
            acc[s] = acc[s] + tmp[s]
        return carry

    lax.fori_loop(1, NS, red, 0)
    pltpu.sync_copy(acc, degs.at[cid, 0, pl.ds(r0, RPT)])


def _sc_degrees(edges):
    zeros1d = jnp.zeros((ROWS,), jnp.float32)
    out_type = jax.ShapeDtypeStruct((NC, 1, ROWS), jnp.float32)
    scratch = [
        pltpu.VMEM((ROWS,), jnp.float32),
        pltpu.VMEM_SHARED((NS, ROWS), jnp.float32),
        pltpu.VMEM((RPT,), jnp.float32),
        pltpu.VMEM((RPT,), jnp.float32),
        pltpu.VMEM((C,), jnp.int32),
    ]
    fn = pl.kernel(_deg_body, out_type=out_type, mesh=_mesh,
                   scratch_types=scratch,
                   compiler_params=pltpu.CompilerParams(
                       needs_layout_passes=False))
    return fn(edges, zeros1d)


def _agg_body(xs, srcg, dstp, zrows, out, agg_sh, rows_v, idx_s, idx_d, sem):
    nch = srcg.shape[2]
    cid = lax.axis_index("c")
    sid = lax.axis_index("s")
    r0 = sid * RPT
    pltpu.sync_copy(zrows, agg_sh.at[pl.ds(r0, RPT)])
    plsc.subcore_barrier()

    def body(j, carry):
        pltpu.sync_copy(srcg.at[cid, sid, j], idx_s)
        pltpu.sync_copy(dstp.at[cid, sid, j], idx_d)
        pltpu.async_copy(xs.at[idx_s], rows_v, sem).wait()
        pltpu.sync_copy(rows_v, agg_sh.at[idx_d], add=True)
        return carry

    lax.fori_loop(0, nch, body, 0)
    plsc.subcore_barrier()
    pltpu.sync_copy(agg_sh.at[pl.ds(r0, RPT)], out.at[cid, pl.ds(r0, RPT)])


def _sc_aggregate(xs, srcg, dstp):
    zrows = jnp.zeros((RPT, D), jnp.float32)
    out_type = jax.ShapeDtypeStruct((NC, ROWS, D), jnp.float32)
    scratch = [
        pltpu.VMEM_SHARED((ROWS, D), jnp.float32),
        pltpu.VMEM((C, D), jnp.float32),
        pltpu.VMEM((C,), jnp.int32),
        pltpu.VMEM((C,), jnp.int32),
        pltpu.SemaphoreType.DMA,
    ]
    fn = pl.kernel(_agg_body, out_type=out_type, mesh=_mesh,
                   scratch_types=scratch)
    return fn(xs, srcg, dstp, zrows)


def _norm_col(deg):
    return jnp.where(deg > 0, lax.rsqrt(jnp.maximum(deg, 1.0)), 0.0)


def _tc_a_body(h_ref, w_ref, dego_ref, xs_ref):
    ns = _norm_col(dego_ref[...])
    x = lax.dot_general(h_ref[...], w_ref[...], (((1,), (1,)), ((), ())),
                        preferred_element_type=jnp.float32)
    xs_ref[...] = x * ns


def _tc_a(h, weights, dego):
    return pl.pallas_call(
        _tc_a_body,
        out_shape=jax.ShapeDtypeStruct((N, D), jnp.float32),
    )(h, weights, dego)


def _layer_post(aggp, degi, W, b, g, beta):
    nd = _norm_col(degi)
    agg = (aggp[0, :N, :] + aggp[1, :N, :]) * nd
    t = lax.dot_general(agg, W, (((1,), (0,)), ((), ())),
                        preferred_element_type=jnp.float32) + b
    m = jnp.mean(t, axis=0, keepdims=True)
    v = jnp.mean((t - m) ** 2, axis=0, keepdims=True)
    t = (t - m) * lax.rsqrt(v + EPS) * g + beta
    return jnp.maximum(t, 0.0)


def _tc_b_body(aggp_ref, dego_ref, degi_ref, W_ref, b_ref, g_ref, beta_ref,
               xs_ref):
    r = _layer_post(aggp_ref[...], degi_ref[...], W_ref[...], b_ref[...],
                    g_ref[...], beta_ref[...])
    xs_ref[...] = r * _norm_col(dego_ref[...])


def _tc_b(aggp, dego, degi, W, b, g, beta):
    return pl.pallas_call(
        _tc_b_body,
        out_shape=jax.ShapeDtypeStruct((N, D), jnp.float32),
    )(aggp, dego, degi, W, b, g, beta)


def _tc_c_body(aggp_ref, degi_ref, W_ref, b_ref, g_ref, beta_ref, gid_ref,
               fcW_ref, fcb_ref, out_ref):
    r = _layer_post(aggp_ref[...], degi_ref[...], W_ref[...], b_ref[...],
                    g_ref[...], beta_ref[...])
    gid = gid_ref[...]
    neg = jnp.float32(-jnp.inf)
    rows = [jnp.max(jnp.where(gid == gg, r, neg), axis=0, keepdims=True)
            for gg in range(G)]
    pooled = jnp.concatenate(rows, axis=0)
    out_ref[...] = lax.dot_general(pooled, fcW_ref[...],
                                   (((1,), (1,)), ((), ())),
                                   preferred_element_type=jnp.float32) \
        + fcb_ref[...]


def _tc_c(aggp, degi, W, b, g, beta, gid, fcW, fcb):
    return pl.pallas_call(
        _tc_c_body,
        out_shape=jax.ShapeDtypeStruct((G, D), jnp.float32),
    )(aggp, degi, W, b, g, beta, gid, fcW, fcb)


def kernel(h, edge_index, graph_ids, weights, W0, b0, g0, beta0,
           W1, b1, g1, beta1, fcW, fcb):
    src = edge_index[0]
    dst = edge_index[1]
    E = src.shape[0]
    ept = E // NW
    nch = -(-ept // C)
    pad = nch * C - ept

    srcm = src.reshape(NW, ept)
    dstm = dst.reshape(NW, ept)
    shape4 = (NC, NS, nch, C)
    srcg = jnp.pad(srcm, ((0, 0), (0, pad))).reshape(shape4)
    srcd = jnp.pad(srcm, ((0, 0), (0, pad)),
                   constant_values=N).reshape(shape4)
    dstp = jnp.pad(dstm, ((0, 0), (0, pad)),
                   constant_values=N).reshape(shape4)

    b0r, g0r, beta0r = b0.reshape(1, -1), g0.reshape(1, -1), beta0.reshape(1, -1)
    b1r, g1r, beta1r = b1.reshape(1, -1), g1.reshape(1, -1), beta1.reshape(1, -1)
    fcbr = fcb.reshape(1, -1)
    gid = graph_ids.reshape(N, 1)

    edges_deg = jnp.stack([srcd.reshape(NS, 2 * nch, C),
                           dstp.reshape(NS, 2 * nch, C)])
    degs = _sc_degrees(edges_deg)
    dego = degs[0, 0, :N].reshape(N, 1)
    degi = degs[1, 0, :N].reshape(N, 1)
    xs1 = _tc_a(h, weights, dego)
    aggp1 = _sc_aggregate(xs1, srcg, dstp)
    xs2 = _tc_b(aggp1, dego, degi, W0, b0r, g0r, beta0r)
    aggp2 = _sc_aggregate(xs2, srcg, dstp)
    out = _tc_c(aggp2, degi, W1, b1r, g1r, beta1r, gid, fcW, fcbr)
    return out
